# baseline (speedup 1.0000x reference)
import functools

import jax
import jax.numpy as jnp
from jax import lax
from jax.experimental import pallas as pl
from jax.experimental.pallas import tpu as pltpu
from jax.experimental.pallas import tpu_sc as plsc

E = 320000
D = 128
H1 = 256
H2 = 128
CHUNK = 128
NW = 32
NBUF = 3
NT = 81
PER_W = NT * CHUNK
E_PAD = NW * PER_W


def _sc_gather(h, src2d, dst2d):
    info = plsc.get_sparse_core_info()
    nc = info.num_cores
    mesh = plsc.VectorSubcoreMesh(core_axis_name="c", subcore_axis_name="s")

    @functools.partial(
        pl.kernel,
        mesh=mesh,
        out_type=(
            jax.ShapeDtypeStruct((E_PAD, D), jnp.float32),
            jax.ShapeDtypeStruct((E_PAD, D), jnp.float32),
        ),
        scratch_types=[
            pltpu.VMEM((PER_W,), jnp.int32),
            pltpu.VMEM((PER_W,), jnp.int32),
            pltpu.VMEM((NBUF, CHUNK, D), jnp.float32),
            pltpu.VMEM((NBUF, CHUNK, D), jnp.float32),
            pltpu.SemaphoreType.DMA,
            pltpu.SemaphoreType.DMA,
            pltpu.SemaphoreType.DMA,
            pltpu.SemaphoreType.DMA,
            pltpu.SemaphoreType.DMA,
            pltpu.SemaphoreType.DMA,
        ],
    )
    def k(h_hbm, src_hbm, dst_hbm, g0_hbm, g1_hbm, idx_s, idx_d, rows_s, rows_d,
          gsem0, gsem1, gsem2, wsem0, wsem1, wsem2):
        gsems = (gsem0, gsem1, gsem2)
        wsems = (wsem0, wsem1, wsem2)
        wid = lax.axis_index("s") * nc + lax.axis_index("c")
        ebase = wid * PER_W

        pltpu.sync_copy(src_hbm.at[wid], idx_s)
        pltpu.sync_copy(dst_hbm.at[wid], idx_d)

        def issue_gather(i, b):
            pltpu.async_copy(h_hbm.at[idx_s.at[pl.ds(i * CHUNK, CHUNK)]],
                             rows_s.at[b], gsems[b])
            pltpu.async_copy(h_hbm.at[idx_d.at[pl.ds(i * CHUNK, CHUNK)]],
                             rows_d.at[b], gsems[b])

        def wait_gather(i, b):
            pltpu.make_async_copy(h_hbm.at[idx_s.at[pl.ds(i * CHUNK, CHUNK)]],
                                  rows_s.at[b], gsems[b]).wait()
            pltpu.make_async_copy(h_hbm.at[idx_d.at[pl.ds(i * CHUNK, CHUNK)]],
                                  rows_d.at[b], gsems[b]).wait()

        def issue_wb(i, b):
            off = ebase + i * CHUNK
            pltpu.async_copy(rows_s.at[b], g0_hbm.at[pl.ds(off, CHUNK)], wsems[b])
            pltpu.async_copy(rows_d.at[b], g1_hbm.at[pl.ds(off, CHUNK)], wsems[b])

        def wait_wb(i, b):
            off = ebase + i * CHUNK
            pltpu.make_async_copy(rows_s.at[b], g0_hbm.at[pl.ds(off, CHUNK)],
                                  wsems[b]).wait()
            pltpu.make_async_copy(rows_d.at[b], g1_hbm.at[pl.ds(off, CHUNK)],
                                  wsems[b]).wait()

        issue_gather(0, 0)
        issue_gather(1, 1)
        wait_gather(0, 0)
        issue_wb(0, 0)
        issue_gather(2, 2)
        wait_gather(1, 1)
        issue_wb(1, 1)

        def body(j, carry):
            for b in range(NBUF):
                i = 3 * j + b
                wait_wb(i - 3, b)
                issue_gather(i, b)
                pb = (b + 2) % 3
                wait_gather(i - 1, pb)
                issue_wb(i - 1, pb)
            return carry

        lax.fori_loop(1, NT // 3, body, 0)

        wait_gather(NT - 1, (NT - 1) % 3)
        issue_wb(NT - 1, (NT - 1) % 3)
        wait_wb(NT - 3, (NT - 3) % 3)
        wait_wb(NT - 2, (NT - 2) % 3)
        wait_wb(NT - 1, (NT - 1) % 3)

    return k(h, src2d, dst2d)


R = 1280


def _mlp_body(g0, g1, w1a, w1b, b1, w2, b2, w3, b3, out):
    z = jnp.dot(g0[...], w1a[...], preferred_element_type=jnp.float32)
    z = z + jnp.dot(g1[...], w1b[...], preferred_element_type=jnp.float32)
    z = jnp.maximum(z + b1[...], 0.0)
    z = jnp.dot(z, w2[...], preferred_element_type=jnp.float32) + b2[...]
    z = jnp.maximum(z, 0.0)
    out[...] = jnp.dot(z, w3[...], preferred_element_type=jnp.float32) + b3[...]


def _tc_mlp(g0, g1, w1a, w1b, b1, w2, b2, w3, b3):
    grid = (E // R,)
    return pl.pallas_call(
        _mlp_body,
        grid=grid,
        in_specs=[
            pl.BlockSpec((R, D), lambda i: (i, 0)),
            pl.BlockSpec((R, D), lambda i: (i, 0)),
            pl.BlockSpec((D, H1), lambda i: (0, 0)),
            pl.BlockSpec((D, H1), lambda i: (0, 0)),
            pl.BlockSpec((1, H1), lambda i: (0, 0)),
            pl.BlockSpec((H1, H2), lambda i: (0, 0)),
            pl.BlockSpec((1, H2), lambda i: (0, 0)),
            pl.BlockSpec((H2, 1), lambda i: (0, 0)),
            pl.BlockSpec((1, 1), lambda i: (0, 0)),
        ],
        out_specs=pl.BlockSpec((R, 1), lambda i: (i, 0)),
        out_shape=jax.ShapeDtypeStruct((E, 1), jnp.float32),
    )(g0, g1, w1a, w1b, b1, w2, b2, w3, b3)


def kernel(h, edge_index, W1, b1, W2, b2, W3, b3):
    idx = edge_index.astype(jnp.int32)
    idx = jnp.pad(idx, ((0, 0), (0, E_PAD - E)))
    src2d = idx[0].reshape(NW, PER_W)
    dst2d = idx[1].reshape(NW, PER_W)
    g0, g1 = _sc_gather(h, src2d, dst2d)
    w1a = W1[:, :D].T.astype(jnp.bfloat16)
    w1b = W1[:, D:].T.astype(jnp.bfloat16)
    w2 = W2.T.astype(jnp.bfloat16)
    w3 = W3.T.astype(jnp.bfloat16)
    return _tc_mlp(
        g0, g1, w1a, w1b,
        b1.reshape(1, H1), w2, b2.reshape(1, H2), w3, b3.reshape(1, 1),
    )


# speedup vs baseline: 1.0005x; 1.0005x over previous
import functools

import jax
import jax.numpy as jnp
from jax import lax
from jax.experimental import pallas as pl
from jax.experimental.pallas import tpu as pltpu
from jax.experimental.pallas import tpu_sc as plsc

E = 320000
D = 128
H1 = 256
H2 = 128
CHUNK = 128
NW = 32
NBUF = 3
NT = 81
PER_W = NT * CHUNK
E_PAD = NW * PER_W


def _sc_gather(h, src2d, dst2d):
    info = plsc.get_sparse_core_info()
    nc = info.num_cores
    mesh = plsc.VectorSubcoreMesh(core_axis_name="c", subcore_axis_name="s")

    @functools.partial(
        pl.kernel,
        mesh=mesh,
        out_type=(
            jax.ShapeDtypeStruct((E_PAD, D), jnp.float32),
            jax.ShapeDtypeStruct((E_PAD, D), jnp.float32),
        ),
        scratch_types=[
            pltpu.VMEM((NBUF, CHUNK), jnp.int32),
            pltpu.VMEM((NBUF, CHUNK), jnp.int32),
            pltpu.VMEM((NBUF, CHUNK, D), jnp.float32),
            pltpu.VMEM((NBUF, CHUNK, D), jnp.float32),
            pltpu.SemaphoreType.DMA,
            pltpu.SemaphoreType.DMA,
            pltpu.SemaphoreType.DMA,
            pltpu.SemaphoreType.DMA,
            pltpu.SemaphoreType.DMA,
            pltpu.SemaphoreType.DMA,
            pltpu.SemaphoreType.DMA,
            pltpu.SemaphoreType.DMA,
            pltpu.SemaphoreType.DMA,
        ],
    )
    def k(h_hbm, src_hbm, dst_hbm, g0_hbm, g1_hbm, idx_s, idx_d, rows_s, rows_d,
          gsem0, gsem1, gsem2, wsem0, wsem1, wsem2, isem0, isem1, isem2):
        gsems = (gsem0, gsem1, gsem2)
        wsems = (wsem0, wsem1, wsem2)
        isems = (isem0, isem1, isem2)
        wid = lax.axis_index("s") * nc + lax.axis_index("c")
        ebase = wid * PER_W

        def issue_idx(i, b):
            off = i * CHUNK
            pltpu.async_copy(src_hbm.at[wid, pl.ds(off, CHUNK)], idx_s.at[b],
                             isems[b])
            pltpu.async_copy(dst_hbm.at[wid, pl.ds(off, CHUNK)], idx_d.at[b],
                             isems[b])

        def wait_idx(i, b):
            off = i * CHUNK
            pltpu.make_async_copy(src_hbm.at[wid, pl.ds(off, CHUNK)],
                                  idx_s.at[b], isems[b]).wait()
            pltpu.make_async_copy(dst_hbm.at[wid, pl.ds(off, CHUNK)],
                                  idx_d.at[b], isems[b]).wait()

        def issue_gather(i, b):
            pltpu.async_copy(h_hbm.at[idx_s.at[b]], rows_s.at[b], gsems[b])
            pltpu.async_copy(h_hbm.at[idx_d.at[b]], rows_d.at[b], gsems[b])

        def wait_gather(i, b):
            pltpu.make_async_copy(h_hbm.at[idx_s.at[b]], rows_s.at[b],
                                  gsems[b]).wait()
            pltpu.make_async_copy(h_hbm.at[idx_d.at[b]], rows_d.at[b],
                                  gsems[b]).wait()

        def issue_wb(i, b):
            off = ebase + i * CHUNK
            pltpu.async_copy(rows_s.at[b], g0_hbm.at[pl.ds(off, CHUNK)], wsems[b])
            pltpu.async_copy(rows_d.at[b], g1_hbm.at[pl.ds(off, CHUNK)], wsems[b])

        def wait_wb(i, b):
            off = ebase + i * CHUNK
            pltpu.make_async_copy(rows_s.at[b], g0_hbm.at[pl.ds(off, CHUNK)],
                                  wsems[b]).wait()
            pltpu.make_async_copy(rows_d.at[b], g1_hbm.at[pl.ds(off, CHUNK)],
                                  wsems[b]).wait()

        issue_idx(0, 0)
        issue_idx(1, 1)
        wait_idx(0, 0)
        issue_gather(0, 0)
        issue_idx(2, 2)
        wait_idx(1, 1)
        issue_gather(1, 1)
        wait_gather(0, 0)
        issue_wb(0, 0)
        issue_idx(3, 0)
        wait_idx(2, 2)
        issue_gather(2, 2)
        wait_gather(1, 1)
        issue_wb(1, 1)
        issue_idx(4, 1)

        def body(j, carry):
            for b in range(NBUF):
                i = 3 * j + b
                wait_wb(i - 3, b)
                wait_idx(i, b)
                issue_gather(i, b)
                pb = (b + 2) % 3
                wait_gather(i - 1, pb)
                issue_wb(i - 1, pb)

                @pl.when(i + 2 < NT)
                def _():
                    issue_idx(i + 2, pb)

            return carry

        lax.fori_loop(1, NT // 3, body, 0)

        wait_gather(NT - 1, (NT - 1) % 3)
        issue_wb(NT - 1, (NT - 1) % 3)
        wait_wb(NT - 3, (NT - 3) % 3)
        wait_wb(NT - 2, (NT - 2) % 3)
        wait_wb(NT - 1, (NT - 1) % 3)

    return k(h, src2d, dst2d)


R = 1280


def _mlp_body(g0, g1, w1a, w1b, b1, w2, b2, w3, b3, out):
    z = jnp.dot(g0[...], w1a[...], preferred_element_type=jnp.float32)
    z = z + jnp.dot(g1[...], w1b[...], preferred_element_type=jnp.float32)
    z = jnp.maximum(z + b1[...], 0.0)
    z = jnp.dot(z, w2[...], preferred_element_type=jnp.float32) + b2[...]
    z = jnp.maximum(z, 0.0)
    out[...] = jnp.dot(z, w3[...], preferred_element_type=jnp.float32) + b3[...]


def _tc_mlp(g0, g1, w1a, w1b, b1, w2, b2, w3, b3):
    grid = (E // R,)
    return pl.pallas_call(
        _mlp_body,
        grid=grid,
        in_specs=[
            pl.BlockSpec((R, D), lambda i: (i, 0)),
            pl.BlockSpec((R, D), lambda i: (i, 0)),
            pl.BlockSpec((D, H1), lambda i: (0, 0)),
            pl.BlockSpec((D, H1), lambda i: (0, 0)),
            pl.BlockSpec((1, H1), lambda i: (0, 0)),
            pl.BlockSpec((H1, H2), lambda i: (0, 0)),
            pl.BlockSpec((1, H2), lambda i: (0, 0)),
            pl.BlockSpec((H2, 1), lambda i: (0, 0)),
            pl.BlockSpec((1, 1), lambda i: (0, 0)),
        ],
        out_specs=pl.BlockSpec((R, 1), lambda i: (i, 0)),
        out_shape=jax.ShapeDtypeStruct((E, 1), jnp.float32),
    )(g0, g1, w1a, w1b, b1, w2, b2, w3, b3)


def kernel(h, edge_index, W1, b1, W2, b2, W3, b3):
    idx = edge_index.astype(jnp.int32)
    idx = jnp.pad(idx, ((0, 0), (0, E_PAD - E)))
    src2d = idx[0].reshape(NW, PER_W)
    dst2d = idx[1].reshape(NW, PER_W)
    g0, g1 = _sc_gather(h, src2d, dst2d)
    w1a = W1[:, :D].T.astype(jnp.bfloat16)
    w1b = W1[:, D:].T.astype(jnp.bfloat16)
    w2 = W2.T.astype(jnp.bfloat16)
    w3 = W3.T.astype(jnp.bfloat16)
    return _tc_mlp(
        g0, g1, w1a, w1b,
        b1.reshape(1, H1), w2, b2.reshape(1, H2), w3, b3.reshape(1, 1),
    )


# speedup vs baseline: 1.1457x; 1.1452x over previous
import functools

import jax
import jax.numpy as jnp
from jax import lax
from jax.experimental import pallas as pl
from jax.experimental.pallas import tpu as pltpu
from jax.experimental.pallas import tpu_sc as plsc

E = 320000
D = 128
H1 = 256
H2 = 128
CHUNK = 128
NW = 32
NBUF = 3
NT = 81
PER_W = NT * CHUNK
E_PAD = NW * PER_W


def _sc_gather(h, src, dst):
    info = plsc.get_sparse_core_info()
    nc = info.num_cores
    mesh = plsc.VectorSubcoreMesh(core_axis_name="c", subcore_axis_name="s")

    @functools.partial(
        pl.kernel,
        mesh=mesh,
        out_type=(
            jax.ShapeDtypeStruct((E_PAD, D), jnp.float32),
            jax.ShapeDtypeStruct((E_PAD, D), jnp.float32),
        ),
        scratch_types=[
            pltpu.VMEM((NBUF, CHUNK), jnp.int32),
            pltpu.VMEM((NBUF, CHUNK), jnp.int32),
            pltpu.VMEM((NBUF, CHUNK, D), jnp.float32),
            pltpu.VMEM((NBUF, CHUNK, D), jnp.float32),
            pltpu.SemaphoreType.DMA,
            pltpu.SemaphoreType.DMA,
            pltpu.SemaphoreType.DMA,
            pltpu.SemaphoreType.DMA,
            pltpu.SemaphoreType.DMA,
            pltpu.SemaphoreType.DMA,
            pltpu.SemaphoreType.DMA,
            pltpu.SemaphoreType.DMA,
            pltpu.SemaphoreType.DMA,
        ],
    )
    def k(h_hbm, src_hbm, dst_hbm, g0_hbm, g1_hbm, idx_s, idx_d, rows_s, rows_d,
          gsem0, gsem1, gsem2, wsem0, wsem1, wsem2, isem0, isem1, isem2):
        gsems = (gsem0, gsem1, gsem2)
        wsems = (wsem0, wsem1, wsem2)
        isems = (isem0, isem1, isem2)
        wid = lax.axis_index("s") * nc + lax.axis_index("c")

        def chunk_off(i):
            return (wid + i * NW) * CHUNK

        def issue_idx(i, b):
            off = chunk_off(i)
            pltpu.async_copy(src_hbm.at[pl.ds(off, CHUNK)], idx_s.at[b],
                             isems[b])
            pltpu.async_copy(dst_hbm.at[pl.ds(off, CHUNK)], idx_d.at[b],
                             isems[b])

        def wait_idx(i, b):
            off = chunk_off(i)
            pltpu.make_async_copy(src_hbm.at[pl.ds(off, CHUNK)],
                                  idx_s.at[b], isems[b]).wait()
            pltpu.make_async_copy(dst_hbm.at[pl.ds(off, CHUNK)],
                                  idx_d.at[b], isems[b]).wait()

        def issue_gather(i, b):
            pltpu.async_copy(h_hbm.at[idx_s.at[b]], rows_s.at[b], gsems[b])
            pltpu.async_copy(h_hbm.at[idx_d.at[b]], rows_d.at[b], gsems[b])

        def wait_gather(i, b):
            pltpu.make_async_copy(h_hbm.at[idx_s.at[b]], rows_s.at[b],
                                  gsems[b]).wait()
            pltpu.make_async_copy(h_hbm.at[idx_d.at[b]], rows_d.at[b],
                                  gsems[b]).wait()

        def issue_wb(i, b):
            off = chunk_off(i)
            pltpu.async_copy(rows_s.at[b], g0_hbm.at[pl.ds(off, CHUNK)], wsems[b])
            pltpu.async_copy(rows_d.at[b], g1_hbm.at[pl.ds(off, CHUNK)], wsems[b])

        def wait_wb(i, b):
            off = chunk_off(i)
            pltpu.make_async_copy(rows_s.at[b], g0_hbm.at[pl.ds(off, CHUNK)],
                                  wsems[b]).wait()
            pltpu.make_async_copy(rows_d.at[b], g1_hbm.at[pl.ds(off, CHUNK)],
                                  wsems[b]).wait()

        issue_idx(0, 0)
        issue_idx(1, 1)
        wait_idx(0, 0)
        issue_gather(0, 0)
        issue_idx(2, 2)
        wait_idx(1, 1)
        issue_gather(1, 1)
        wait_gather(0, 0)
        issue_wb(0, 0)
        issue_idx(3, 0)
        wait_idx(2, 2)
        issue_gather(2, 2)
        wait_gather(1, 1)
        issue_wb(1, 1)
        issue_idx(4, 1)

        def body(j, carry):
            for b in range(NBUF):
                i = 3 * j + b
                wait_wb(i - 3, b)
                wait_idx(i, b)
                issue_gather(i, b)
                pb = (b + 2) % 3
                wait_gather(i - 1, pb)
                issue_wb(i - 1, pb)

                @pl.when(i + 2 < NT)
                def _():
                    issue_idx(i + 2, pb)

            return carry

        lax.fori_loop(1, NT // 3, body, 0)

        wait_gather(NT - 1, (NT - 1) % 3)
        issue_wb(NT - 1, (NT - 1) % 3)
        wait_wb(NT - 3, (NT - 3) % 3)
        wait_wb(NT - 2, (NT - 2) % 3)
        wait_wb(NT - 1, (NT - 1) % 3)

    return k(h, src, dst)


R = 1280


def _mlp_body(g0, g1, w1a, w1b, b1, w2, b2, w3, b3, out):
    z = jnp.dot(g0[...], w1a[...], preferred_element_type=jnp.float32)
    z = z + jnp.dot(g1[...], w1b[...], preferred_element_type=jnp.float32)
    z = jnp.maximum(z + b1[...], 0.0)
    z = jnp.dot(z, w2[...], preferred_element_type=jnp.float32) + b2[...]
    z = jnp.maximum(z, 0.0)
    out[...] = jnp.dot(z, w3[...], preferred_element_type=jnp.float32) + b3[...]


def _tc_mlp(g0, g1, w1a, w1b, b1, w2, b2, w3, b3):
    grid = (E // R,)
    return pl.pallas_call(
        _mlp_body,
        grid=grid,
        in_specs=[
            pl.BlockSpec((R, D), lambda i: (i, 0)),
            pl.BlockSpec((R, D), lambda i: (i, 0)),
            pl.BlockSpec((D, H1), lambda i: (0, 0)),
            pl.BlockSpec((D, H1), lambda i: (0, 0)),
            pl.BlockSpec((1, H1), lambda i: (0, 0)),
            pl.BlockSpec((H1, H2), lambda i: (0, 0)),
            pl.BlockSpec((1, H2), lambda i: (0, 0)),
            pl.BlockSpec((H2, 1), lambda i: (0, 0)),
            pl.BlockSpec((1, 1), lambda i: (0, 0)),
        ],
        out_specs=pl.BlockSpec((R, 1), lambda i: (i, 0)),
        out_shape=jax.ShapeDtypeStruct((E, 1), jnp.float32),
    )(g0, g1, w1a, w1b, b1, w2, b2, w3, b3)


def kernel(h, edge_index, W1, b1, W2, b2, W3, b3):
    idx = edge_index.astype(jnp.int32)
    idx = jnp.pad(idx, ((0, 0), (0, E_PAD - E)))
    g0, g1 = _sc_gather(h, idx[0], idx[1])
    w1a = W1[:, :D].T.astype(jnp.bfloat16)
    w1b = W1[:, D:].T.astype(jnp.bfloat16)
    w2 = W2.T.astype(jnp.bfloat16)
    w3 = W3.T.astype(jnp.bfloat16)
    return _tc_mlp(
        g0, g1, w1a, w1b,
        b1.reshape(1, H1), w2, b2.reshape(1, H2), w3, b3.reshape(1, 1),
    )


# speedup vs baseline: 1.4009x; 1.2228x over previous
import functools

import jax
import jax.numpy as jnp
from jax import lax
from jax.experimental import pallas as pl
from jax.experimental.pallas import tpu as pltpu
from jax.experimental.pallas import tpu_sc as plsc

E = 320000
D = 128
H1 = 256
H2 = 128
CHUNK = 128
NW = 32
NBUF = 2
NT = 80
PER_W = NT * CHUNK
E_PAD = NW * PER_W


def _sc_gather(h, src, dst):
    info = plsc.get_sparse_core_info()
    nc = info.num_cores
    mesh = plsc.VectorSubcoreMesh(core_axis_name="c", subcore_axis_name="s")

    @functools.partial(
        pl.kernel,
        mesh=mesh,
        out_type=(
            jax.ShapeDtypeStruct((E_PAD, D), jnp.float32),
            jax.ShapeDtypeStruct((E_PAD, D), jnp.float32),
        ),
        scratch_types=[
            pltpu.VMEM((NBUF, CHUNK), jnp.int32),
            pltpu.VMEM((NBUF, CHUNK), jnp.int32),
            pltpu.VMEM((NBUF, CHUNK, D), jnp.float32),
            pltpu.VMEM((NBUF, CHUNK, D), jnp.float32),
            pltpu.SemaphoreType.DMA,
            pltpu.SemaphoreType.DMA,
        ],
    )
    def k(h_hbm, src_hbm, dst_hbm, g0_hbm, g1_hbm, idx_s, idx_d, rows_s, rows_d,
          gsem0, gsem1):
        gsems = (gsem0, gsem1)
        wid = lax.axis_index("s") * nc + lax.axis_index("c")

        def chunk_off(i):
            return (wid + i * NW) * CHUNK

        def load_idx(i, b):
            off = chunk_off(i)
            pltpu.sync_copy(src_hbm.at[pl.ds(off, CHUNK)], idx_s.at[b])
            pltpu.sync_copy(dst_hbm.at[pl.ds(off, CHUNK)], idx_d.at[b])

        def issue_gather(b):
            pltpu.async_copy(h_hbm.at[idx_s.at[b]], rows_s.at[b], gsems[b])
            pltpu.async_copy(h_hbm.at[idx_d.at[b]], rows_d.at[b], gsems[b])

        def wait_gather(b):
            pltpu.make_async_copy(h_hbm.at[idx_s.at[b]], rows_s.at[b],
                                  gsems[b]).wait()
            pltpu.make_async_copy(h_hbm.at[idx_d.at[b]], rows_d.at[b],
                                  gsems[b]).wait()

        def write_back(i, b):
            off = chunk_off(i)
            pltpu.sync_copy(rows_s.at[b], g0_hbm.at[pl.ds(off, CHUNK)])
            pltpu.sync_copy(rows_d.at[b], g1_hbm.at[pl.ds(off, CHUNK)])

        load_idx(0, 0)
        issue_gather(0)

        def body(j, carry):
            for b in range(NBUF):
                i = 2 * j + b
                nb = 1 - b

                @pl.when(i + 1 < NT)
                def _():
                    load_idx(i + 1, nb)
                    issue_gather(nb)

                wait_gather(b)
                write_back(i, b)
            return carry

        lax.fori_loop(0, NT // 2, body, 0)

    return k(h, src, dst)


R = 1280


def _mlp_body(g0, g1, w1a, w1b, b1, w2, b2, w3, b3, out):
    z = jnp.dot(g0[...], w1a[...], preferred_element_type=jnp.float32)
    z = z + jnp.dot(g1[...], w1b[...], preferred_element_type=jnp.float32)
    z = jnp.maximum(z + b1[...], 0.0)
    z = jnp.dot(z, w2[...], preferred_element_type=jnp.float32) + b2[...]
    z = jnp.maximum(z, 0.0)
    out[...] = jnp.dot(z, w3[...], preferred_element_type=jnp.float32) + b3[...]


def _tc_mlp(g0, g1, w1a, w1b, b1, w2, b2, w3, b3):
    grid = (E // R,)
    return pl.pallas_call(
        _mlp_body,
        grid=grid,
        in_specs=[
            pl.BlockSpec((R, D), lambda i: (i, 0)),
            pl.BlockSpec((R, D), lambda i: (i, 0)),
            pl.BlockSpec((D, H1), lambda i: (0, 0)),
            pl.BlockSpec((D, H1), lambda i: (0, 0)),
            pl.BlockSpec((1, H1), lambda i: (0, 0)),
            pl.BlockSpec((H1, H2), lambda i: (0, 0)),
            pl.BlockSpec((1, H2), lambda i: (0, 0)),
            pl.BlockSpec((H2, 1), lambda i: (0, 0)),
            pl.BlockSpec((1, 1), lambda i: (0, 0)),
        ],
        out_specs=pl.BlockSpec((R, 1), lambda i: (i, 0)),
        out_shape=jax.ShapeDtypeStruct((E, 1), jnp.float32),
    )(g0, g1, w1a, w1b, b1, w2, b2, w3, b3)


def kernel(h, edge_index, W1, b1, W2, b2, W3, b3):
    idx = edge_index.astype(jnp.int32)
    idx = jnp.pad(idx, ((0, 0), (0, E_PAD - E)))
    g0, g1 = _sc_gather(h, idx[0], idx[1])
    w1a = W1[:, :D].T.astype(jnp.bfloat16)
    w1b = W1[:, D:].T.astype(jnp.bfloat16)
    w2 = W2.T.astype(jnp.bfloat16)
    w3 = W3.T.astype(jnp.bfloat16)
    return _tc_mlp(
        g0, g1, w1a, w1b,
        b1.reshape(1, H1), w2, b2.reshape(1, H2), w3, b3.reshape(1, 1),
    )


# speedup vs baseline: 2.5370x; 1.8109x over previous
import functools

import jax
import jax.numpy as jnp
from jax import lax
from jax.experimental import pallas as pl
from jax.experimental.pallas import tpu as pltpu
from jax.experimental.pallas import tpu_sc as plsc

E = 320000
D = 128
H1 = 256
H2 = 128
CHUNK = 128
NCHUNK = E // CHUNK
NW = 32
NT = NCHUNK // NW
NREM = NCHUNK % NW


def _sc_gather(h, src, dst):
    info = plsc.get_sparse_core_info()
    nc = info.num_cores
    mesh = plsc.VectorSubcoreMesh(core_axis_name="c", subcore_axis_name="s")

    @functools.partial(
        pl.kernel,
        mesh=mesh,
        out_type=(
            jax.ShapeDtypeStruct((E, D), jnp.float32),
            jax.ShapeDtypeStruct((E, D), jnp.float32),
        ),
        scratch_types=[
            pltpu.VMEM((CHUNK,), jnp.int32),
            pltpu.VMEM((CHUNK,), jnp.int32),
            pltpu.VMEM((CHUNK,), jnp.int32),
            pltpu.VMEM((CHUNK,), jnp.int32),
            pltpu.VMEM((CHUNK, D), jnp.float32),
            pltpu.VMEM((CHUNK, D), jnp.float32),
            pltpu.VMEM((CHUNK, D), jnp.float32),
            pltpu.VMEM((CHUNK, D), jnp.float32),
            pltpu.SemaphoreType.DMA,
            pltpu.SemaphoreType.DMA,
        ],
    )
    def k(h_hbm, src_hbm, dst_hbm, g0_hbm, g1_hbm,
          idx_s0, idx_d0, idx_s1, idx_d1,
          rows_s0, rows_d0, rows_s1, rows_d1, gsem0, gsem1):
        slots = ((idx_s0, idx_d0, rows_s0, rows_d0, gsem0),
                 (idx_s1, idx_d1, rows_s1, rows_d1, gsem1))
        wid = lax.axis_index("s") * nc + lax.axis_index("c")

        def chunk_off(i):
            return (wid + i * NW) * CHUNK

        def load_idx(i, b):
            idx_s, idx_d = slots[b][0], slots[b][1]
            off = chunk_off(i)
            pltpu.sync_copy(src_hbm.at[pl.ds(off, CHUNK)], idx_s)
            pltpu.sync_copy(dst_hbm.at[pl.ds(off, CHUNK)], idx_d)

        def issue_gather(b):
            idx_s, idx_d, rows_s, rows_d, sem = slots[b]
            pltpu.async_copy(h_hbm.at[idx_s], rows_s, sem)
            pltpu.async_copy(h_hbm.at[idx_d], rows_d, sem)

        def wait_gather(b):
            idx_s, idx_d, rows_s, rows_d, sem = slots[b]
            pltpu.make_async_copy(h_hbm.at[idx_s], rows_s, sem).wait()
            pltpu.make_async_copy(h_hbm.at[idx_d], rows_d, sem).wait()

        def write_back(i, b):
            rows_s, rows_d = slots[b][2], slots[b][3]
            off = chunk_off(i)
            pltpu.sync_copy(rows_s, g0_hbm.at[pl.ds(off, CHUNK)])
            pltpu.sync_copy(rows_d, g1_hbm.at[pl.ds(off, CHUNK)])

        load_idx(0, 0)
        issue_gather(0)

        def body(j, carry):
            for b in range(2):
                i = 2 * j + b
                nb = 1 - b

                @pl.when(i + 1 < NT)
                def _():
                    load_idx(i + 1, nb)
                    issue_gather(nb)

                wait_gather(b)
                write_back(i, b)
            return carry

        lax.fori_loop(0, NT // 2, body, 0)

        @pl.when(wid < NREM)
        def _():
            load_idx(NT, 0)
            issue_gather(0)
            wait_gather(0)
            write_back(NT, 0)

    return k(h, src, dst)


R = 1280


def _mlp_body(g0, g1, w1a, w1b, b1, w2, b2, w3, b3, out):
    z = jnp.dot(g0[...], w1a[...], preferred_element_type=jnp.float32)
    z = z + jnp.dot(g1[...], w1b[...], preferred_element_type=jnp.float32)
    z = jnp.maximum(z + b1[...], 0.0)
    z = jnp.dot(z, w2[...], preferred_element_type=jnp.float32) + b2[...]
    z = jnp.maximum(z, 0.0)
    out[...] = jnp.dot(z, w3[...], preferred_element_type=jnp.float32) + b3[...]


def _tc_mlp(g0, g1, w1a, w1b, b1, w2, b2, w3, b3):
    grid = (E // R,)
    return pl.pallas_call(
        _mlp_body,
        grid=grid,
        in_specs=[
            pl.BlockSpec((R, D), lambda i: (i, 0)),
            pl.BlockSpec((R, D), lambda i: (i, 0)),
            pl.BlockSpec((D, H1), lambda i: (0, 0)),
            pl.BlockSpec((D, H1), lambda i: (0, 0)),
            pl.BlockSpec((1, H1), lambda i: (0, 0)),
            pl.BlockSpec((H1, H2), lambda i: (0, 0)),
            pl.BlockSpec((1, H2), lambda i: (0, 0)),
            pl.BlockSpec((H2, 1), lambda i: (0, 0)),
            pl.BlockSpec((1, 1), lambda i: (0, 0)),
        ],
        out_specs=pl.BlockSpec((R, 1), lambda i: (i, 0)),
        out_shape=jax.ShapeDtypeStruct((E, 1), jnp.float32),
    )(g0, g1, w1a, w1b, b1, w2, b2, w3, b3)


def kernel(h, edge_index, W1, b1, W2, b2, W3, b3):
    idx = edge_index.astype(jnp.int32)
    g0, g1 = _sc_gather(h, idx[0], idx[1])
    w1a = W1[:, :D].T.astype(jnp.bfloat16)
    w1b = W1[:, D:].T.astype(jnp.bfloat16)
    w2 = W2.T.astype(jnp.bfloat16)
    w3 = W3.T.astype(jnp.bfloat16)
    return _tc_mlp(
        g0, g1, w1a, w1b,
        b1.reshape(1, H1), w2, b2.reshape(1, H2), w3, b3.reshape(1, 1),
    )


# speedup vs baseline: 2.7709x; 1.0922x over previous
import functools

import jax
import jax.numpy as jnp
from jax import lax
from jax.experimental import pallas as pl
from jax.experimental.pallas import tpu as pltpu
from jax.experimental.pallas import tpu_sc as plsc

E = 320000
D = 128
H1 = 256
H2 = 128
CHUNK = 128
NCHUNK = E // CHUNK
NW = 32
NT = NCHUNK // NW
NREM = NCHUNK % NW


def _make_sc_gather(nchunks):
    ne = nchunks * CHUNK
    nt = nchunks // NW
    nrem = nchunks % NW
    nt_even = 2 * (nt // 2)
    info = plsc.get_sparse_core_info()
    nc = info.num_cores
    mesh = plsc.VectorSubcoreMesh(core_axis_name="c", subcore_axis_name="s")

    @functools.partial(
        pl.kernel,
        mesh=mesh,
        out_type=(
            jax.ShapeDtypeStruct((ne, D), jnp.float32),
            jax.ShapeDtypeStruct((ne, D), jnp.float32),
        ),
        scratch_types=[
            pltpu.VMEM((CHUNK,), jnp.int32),
            pltpu.VMEM((CHUNK,), jnp.int32),
            pltpu.VMEM((CHUNK,), jnp.int32),
            pltpu.VMEM((CHUNK,), jnp.int32),
            pltpu.VMEM((CHUNK, D), jnp.float32),
            pltpu.VMEM((CHUNK, D), jnp.float32),
            pltpu.VMEM((CHUNK, D), jnp.float32),
            pltpu.VMEM((CHUNK, D), jnp.float32),
            pltpu.SemaphoreType.DMA,
            pltpu.SemaphoreType.DMA,
        ],
    )
    def k(h_hbm, src_hbm, dst_hbm, g0_hbm, g1_hbm,
          idx_s0, idx_d0, idx_s1, idx_d1,
          rows_s0, rows_d0, rows_s1, rows_d1, gsem0, gsem1):
        slots = ((idx_s0, idx_d0, rows_s0, rows_d0, gsem0),
                 (idx_s1, idx_d1, rows_s1, rows_d1, gsem1))
        wid = lax.axis_index("s") * nc + lax.axis_index("c")

        def chunk_off(i):
            return (wid + i * NW) * CHUNK

        def load_idx(i, b):
            idx_s, idx_d = slots[b][0], slots[b][1]
            off = chunk_off(i)
            pltpu.sync_copy(src_hbm.at[pl.ds(off, CHUNK)], idx_s)
            pltpu.sync_copy(dst_hbm.at[pl.ds(off, CHUNK)], idx_d)

        def issue_gather(b):
            idx_s, idx_d, rows_s, rows_d, sem = slots[b]
            pltpu.async_copy(h_hbm.at[idx_s], rows_s, sem)
            pltpu.async_copy(h_hbm.at[idx_d], rows_d, sem)

        def wait_gather(b):
            idx_s, idx_d, rows_s, rows_d, sem = slots[b]
            pltpu.make_async_copy(h_hbm.at[idx_s], rows_s, sem).wait()
            pltpu.make_async_copy(h_hbm.at[idx_d], rows_d, sem).wait()

        def write_back(i, b):
            rows_s, rows_d = slots[b][2], slots[b][3]
            off = chunk_off(i)
            pltpu.sync_copy(rows_s, g0_hbm.at[pl.ds(off, CHUNK)])
            pltpu.sync_copy(rows_d, g1_hbm.at[pl.ds(off, CHUNK)])

        load_idx(0, 0)
        issue_gather(0)

        def body(j, carry):
            for b in range(2):
                i = 2 * j + b
                nb = 1 - b

                @pl.when(i + 1 < nt_even)
                def _():
                    load_idx(i + 1, nb)
                    issue_gather(nb)

                wait_gather(b)
                write_back(i, b)
            return carry

        lax.fori_loop(0, nt // 2, body, 0)

        for ii in range(nt_even, nt):
            load_idx(ii, 0)
            issue_gather(0)
            wait_gather(0)
            write_back(ii, 0)

        if nrem:
            @pl.when(wid < nrem)
            def _():
                load_idx(nt, 0)
                issue_gather(0)
                wait_gather(0)
                write_back(nt, 0)

    def call(h, src, dst):
        return k(h, src, dst)

    return call


R = 1280


def _mlp_body(g0, g1, w1a, w1b, b1, w2, b2, w3, b3, out):
    z = jnp.dot(g0[...], w1a[...], preferred_element_type=jnp.float32)
    z = z + jnp.dot(g1[...], w1b[...], preferred_element_type=jnp.float32)
    z = jnp.maximum(z + b1[...], 0.0)
    z = jnp.dot(z, w2[...], preferred_element_type=jnp.float32) + b2[...]
    z = jnp.maximum(z, 0.0)
    out[...] = jnp.dot(z, w3[...], preferred_element_type=jnp.float32) + b3[...]


def _tc_mlp(g0, g1, w1a, w1b, b1, w2, b2, w3, b3):
    n = g0.shape[0]
    grid = (n // R,)
    return pl.pallas_call(
        _mlp_body,
        grid=grid,
        in_specs=[
            pl.BlockSpec((R, D), lambda i: (i, 0)),
            pl.BlockSpec((R, D), lambda i: (i, 0)),
            pl.BlockSpec((D, H1), lambda i: (0, 0)),
            pl.BlockSpec((D, H1), lambda i: (0, 0)),
            pl.BlockSpec((1, H1), lambda i: (0, 0)),
            pl.BlockSpec((H1, H2), lambda i: (0, 0)),
            pl.BlockSpec((1, H2), lambda i: (0, 0)),
            pl.BlockSpec((H2, 1), lambda i: (0, 0)),
            pl.BlockSpec((1, 1), lambda i: (0, 0)),
        ],
        out_specs=pl.BlockSpec((R, 1), lambda i: (i, 0)),
        out_shape=jax.ShapeDtypeStruct((n, 1), jnp.float32),
    )(g0, g1, w1a, w1b, b1, w2, b2, w3, b3)


NSLICE = 2
ES = E // NSLICE
_gather_slice = _make_sc_gather(ES // CHUNK)


def kernel(h, edge_index, W1, b1, W2, b2, W3, b3):
    idx = edge_index.astype(jnp.int32)
    w1a = W1[:, :D].T.astype(jnp.bfloat16)
    w1b = W1[:, D:].T.astype(jnp.bfloat16)
    w2 = W2.T.astype(jnp.bfloat16)
    w3 = W3.T.astype(jnp.bfloat16)
    outs = []
    for s in range(NSLICE):
        lo = s * ES
        g0, g1 = _gather_slice(h, idx[0, lo:lo + ES], idx[1, lo:lo + ES])
        outs.append(_tc_mlp(
            g0, g1, w1a, w1b,
            b1.reshape(1, H1), w2, b2.reshape(1, H2), w3, b3.reshape(1, 1),
        ))
    return jnp.concatenate(outs, axis=0)


# speedup vs baseline: 3.0162x; 1.0885x over previous
import functools

import jax
import jax.numpy as jnp
from jax import lax
from jax.experimental import pallas as pl
from jax.experimental.pallas import tpu as pltpu
from jax.experimental.pallas import tpu_sc as plsc

E = 320000
D = 128
H1 = 256
H2 = 128
CHUNK = 128
NCHUNK = E // CHUNK
NW = 32
NT = NCHUNK // NW
NREM = NCHUNK % NW


def _make_sc_gather(nchunks):
    ne = nchunks * CHUNK
    nt = nchunks // NW
    nrem = nchunks % NW
    nt_even = 2 * (nt // 2)
    info = plsc.get_sparse_core_info()
    nc = info.num_cores
    mesh = plsc.VectorSubcoreMesh(core_axis_name="c", subcore_axis_name="s")

    @functools.partial(
        pl.kernel,
        mesh=mesh,
        out_type=(
            jax.ShapeDtypeStruct((ne, D), jnp.float32),
            jax.ShapeDtypeStruct((ne, D), jnp.float32),
        ),
        scratch_types=[
            pltpu.VMEM((CHUNK,), jnp.int32),
            pltpu.VMEM((CHUNK,), jnp.int32),
            pltpu.VMEM((CHUNK,), jnp.int32),
            pltpu.VMEM((CHUNK,), jnp.int32),
            pltpu.VMEM((CHUNK, D), jnp.float32),
            pltpu.VMEM((CHUNK, D), jnp.float32),
            pltpu.VMEM((CHUNK, D), jnp.float32),
            pltpu.VMEM((CHUNK, D), jnp.float32),
            pltpu.SemaphoreType.DMA,
            pltpu.SemaphoreType.DMA,
        ],
    )
    def k(h_hbm, src_hbm, dst_hbm, g0_hbm, g1_hbm,
          idx_s0, idx_d0, idx_s1, idx_d1,
          rows_s0, rows_d0, rows_s1, rows_d1, gsem0, gsem1):
        slots = ((idx_s0, idx_d0, rows_s0, rows_d0, gsem0),
                 (idx_s1, idx_d1, rows_s1, rows_d1, gsem1))
        wid = lax.axis_index("s") * nc + lax.axis_index("c")

        def chunk_off(i):
            return (wid + i * NW) * CHUNK

        def load_idx(i, b):
            idx_s, idx_d = slots[b][0], slots[b][1]
            off = chunk_off(i)
            pltpu.sync_copy(src_hbm.at[pl.ds(off, CHUNK)], idx_s)
            pltpu.sync_copy(dst_hbm.at[pl.ds(off, CHUNK)], idx_d)

        def issue_gather(b):
            idx_s, idx_d, rows_s, rows_d, sem = slots[b]
            pltpu.async_copy(h_hbm.at[idx_s], rows_s, sem)
            pltpu.async_copy(h_hbm.at[idx_d], rows_d, sem)

        def wait_gather(b):
            idx_s, idx_d, rows_s, rows_d, sem = slots[b]
            pltpu.make_async_copy(h_hbm.at[idx_s], rows_s, sem).wait()
            pltpu.make_async_copy(h_hbm.at[idx_d], rows_d, sem).wait()

        def write_back(i, b):
            rows_s, rows_d = slots[b][2], slots[b][3]
            off = chunk_off(i)
            pltpu.sync_copy(rows_s, g0_hbm.at[pl.ds(off, CHUNK)])
            pltpu.sync_copy(rows_d, g1_hbm.at[pl.ds(off, CHUNK)])

        load_idx(0, 0)
        issue_gather(0)

        def body(j, carry):
            for b in range(2):
                i = 2 * j + b
                nb = 1 - b

                @pl.when(i + 1 < nt_even)
                def _():
                    load_idx(i + 1, nb)
                    issue_gather(nb)

                wait_gather(b)
                write_back(i, b)
            return carry

        lax.fori_loop(0, nt // 2, body, 0)

        for ii in range(nt_even, nt):
            load_idx(ii, 0)
            issue_gather(0)
            wait_gather(0)
            write_back(ii, 0)

        if nrem:
            @pl.when(wid < nrem)
            def _():
                load_idx(nt, 0)
                issue_gather(0)
                wait_gather(0)
                write_back(nt, 0)

    def call(h, src, dst):
        return k(h, src, dst)

    return call


R = 2000


def _mlp_body(g0, g1, w1a, w1b, b1, w2, b2, w3, b3, out):
    z = jnp.dot(g0[...], w1a[...], preferred_element_type=jnp.float32)
    z = z + jnp.dot(g1[...], w1b[...], preferred_element_type=jnp.float32)
    z = jnp.maximum(z + b1[...], 0.0)
    z = jnp.dot(z, w2[...], preferred_element_type=jnp.float32) + b2[...]
    z = jnp.maximum(z, 0.0)
    out[...] = jnp.dot(z, w3[...], preferred_element_type=jnp.float32) + b3[...]


def _tc_mlp(g0, g1, w1a, w1b, b1, w2, b2, w3, b3):
    n = g0.shape[0]
    grid = (n // R,)
    return pl.pallas_call(
        _mlp_body,
        grid=grid,
        in_specs=[
            pl.BlockSpec((R, D), lambda i: (i, 0)),
            pl.BlockSpec((R, D), lambda i: (i, 0)),
            pl.BlockSpec((D, H1), lambda i: (0, 0)),
            pl.BlockSpec((D, H1), lambda i: (0, 0)),
            pl.BlockSpec((1, H1), lambda i: (0, 0)),
            pl.BlockSpec((H1, H2), lambda i: (0, 0)),
            pl.BlockSpec((1, H2), lambda i: (0, 0)),
            pl.BlockSpec((H2, 1), lambda i: (0, 0)),
            pl.BlockSpec((1, 1), lambda i: (0, 0)),
        ],
        out_specs=pl.BlockSpec((R, 1), lambda i: (i, 0)),
        out_shape=jax.ShapeDtypeStruct((n, 1), jnp.float32),
    )(g0, g1, w1a, w1b, b1, w2, b2, w3, b3)


NSLICE = 2
ES = E // NSLICE
_gather_slice = _make_sc_gather(ES // CHUNK)


def kernel(h, edge_index, W1, b1, W2, b2, W3, b3):
    idx = edge_index.astype(jnp.int32)
    w1a = W1[:, :D].T.astype(jnp.bfloat16)
    w1b = W1[:, D:].T.astype(jnp.bfloat16)
    w2 = W2.T.astype(jnp.bfloat16)
    w3 = W3.T.astype(jnp.bfloat16)
    outs = []
    for s in range(NSLICE):
        lo = s * ES
        g0, g1 = _gather_slice(h, idx[0, lo:lo + ES], idx[1, lo:lo + ES])
        outs.append(_tc_mlp(
            g0, g1, w1a, w1b,
            b1.reshape(1, H1), w2, b2.reshape(1, H2), w3, b3.reshape(1, 1),
        ))
    return jnp.concatenate(outs, axis=0)


# speedup vs baseline: 3.0797x; 1.0210x over previous
import functools

import jax
import jax.numpy as jnp
from jax import lax
from jax.experimental import pallas as pl
from jax.experimental.pallas import tpu as pltpu
from jax.experimental.pallas import tpu_sc as plsc

E = 320000
D = 128
H1 = 256
H2 = 128
CHUNK = 128
NCHUNK = E // CHUNK
NW = 32
NT = NCHUNK // NW
NREM = NCHUNK % NW


def _make_sc_gather(nchunks):
    ne = nchunks * CHUNK
    nt = nchunks // NW
    nrem = nchunks % NW
    nt_even = 2 * (nt // 2)
    info = plsc.get_sparse_core_info()
    nc = info.num_cores
    mesh = plsc.VectorSubcoreMesh(core_axis_name="c", subcore_axis_name="s")

    @functools.partial(
        pl.kernel,
        mesh=mesh,
        out_type=(
            jax.ShapeDtypeStruct((ne, D), jnp.float32),
            jax.ShapeDtypeStruct((ne, D), jnp.float32),
        ),
        scratch_types=[
            pltpu.VMEM((CHUNK,), jnp.int32),
            pltpu.VMEM((CHUNK,), jnp.int32),
            pltpu.VMEM((CHUNK,), jnp.int32),
            pltpu.VMEM((CHUNK,), jnp.int32),
            pltpu.VMEM((CHUNK, D), jnp.float32),
            pltpu.VMEM((CHUNK, D), jnp.float32),
            pltpu.VMEM((CHUNK, D), jnp.float32),
            pltpu.VMEM((CHUNK, D), jnp.float32),
            pltpu.SemaphoreType.DMA,
            pltpu.SemaphoreType.DMA,
        ],
    )
    def k(h_hbm, src_hbm, dst_hbm, g0_hbm, g1_hbm,
          idx_s0, idx_d0, idx_s1, idx_d1,
          rows_s0, rows_d0, rows_s1, rows_d1, gsem0, gsem1):
        slots = ((idx_s0, idx_d0, rows_s0, rows_d0, gsem0),
                 (idx_s1, idx_d1, rows_s1, rows_d1, gsem1))
        wid = lax.axis_index("s") * nc + lax.axis_index("c")

        def chunk_off(i):
            return (wid + i * NW) * CHUNK

        def load_idx(i, b):
            idx_s, idx_d = slots[b][0], slots[b][1]
            off = chunk_off(i)
            pltpu.sync_copy(src_hbm.at[pl.ds(off, CHUNK)], idx_s)
            pltpu.sync_copy(dst_hbm.at[pl.ds(off, CHUNK)], idx_d)

        def issue_gather(b):
            idx_s, idx_d, rows_s, rows_d, sem = slots[b]
            pltpu.async_copy(h_hbm.at[idx_s], rows_s, sem)
            pltpu.async_copy(h_hbm.at[idx_d], rows_d, sem)

        def wait_gather(b):
            idx_s, idx_d, rows_s, rows_d, sem = slots[b]
            pltpu.make_async_copy(h_hbm.at[idx_s], rows_s, sem).wait()
            pltpu.make_async_copy(h_hbm.at[idx_d], rows_d, sem).wait()

        def write_back(i, b):
            rows_s, rows_d = slots[b][2], slots[b][3]
            off = chunk_off(i)
            pltpu.sync_copy(rows_s, g0_hbm.at[pl.ds(off, CHUNK)])
            pltpu.sync_copy(rows_d, g1_hbm.at[pl.ds(off, CHUNK)])

        load_idx(0, 0)
        issue_gather(0)

        def body(j, carry):
            for b in range(2):
                i = 2 * j + b
                nb = 1 - b

                @pl.when(i + 1 < nt_even)
                def _():
                    load_idx(i + 1, nb)
                    issue_gather(nb)

                wait_gather(b)
                write_back(i, b)
            return carry

        lax.fori_loop(0, nt // 2, body, 0)

        for ii in range(nt_even, nt):
            load_idx(ii, 0)
            issue_gather(0)
            wait_gather(0)
            write_back(ii, 0)

        if nrem:
            @pl.when(wid < nrem)
            def _():
                load_idx(nt, 0)
                issue_gather(0)
                wait_gather(0)
                write_back(nt, 0)

    def call(h, src, dst):
        return k(h, src, dst)

    return call


R = 2000


def _mlp_body(g0, g1, w1a, w1b, b1, w2, b2, w3, b3, out):
    z = jnp.dot(g0[...], w1a[...], preferred_element_type=jnp.float32)
    z = z + jnp.dot(g1[...], w1b[...], preferred_element_type=jnp.float32)
    z = jnp.maximum(z + b1[...], 0.0)
    z = jnp.dot(z, w2[...], preferred_element_type=jnp.float32) + b2[...]
    z = jnp.maximum(z, 0.0)
    out[...] = jnp.dot(z, w3[...], preferred_element_type=jnp.float32) + b3[...]


def _tc_mlp(g0, g1, w1a, w1b, b1, w2, b2, w3, b3):
    n = g0.shape[0]
    grid = (n // R,)
    return pl.pallas_call(
        _mlp_body,
        grid=grid,
        in_specs=[
            pl.BlockSpec((R, D), lambda i: (i, 0)),
            pl.BlockSpec((R, D), lambda i: (i, 0)),
            pl.BlockSpec((D, H1), lambda i: (0, 0)),
            pl.BlockSpec((D, H1), lambda i: (0, 0)),
            pl.BlockSpec((1, H1), lambda i: (0, 0)),
            pl.BlockSpec((H1, H2), lambda i: (0, 0)),
            pl.BlockSpec((1, H2), lambda i: (0, 0)),
            pl.BlockSpec((H2, 1), lambda i: (0, 0)),
            pl.BlockSpec((1, 1), lambda i: (0, 0)),
        ],
        out_specs=pl.BlockSpec((R, 1), lambda i: (i, 0)),
        out_shape=jax.ShapeDtypeStruct((n, 1), jnp.float32),
    )(g0, g1, w1a, w1b, b1, w2, b2, w3, b3)


NSLICE = 4
ES = E // NSLICE
_gather_slice = _make_sc_gather(ES // CHUNK)


def kernel(h, edge_index, W1, b1, W2, b2, W3, b3):
    idx = edge_index.astype(jnp.int32)
    w1a = W1[:, :D].T.astype(jnp.bfloat16)
    w1b = W1[:, D:].T.astype(jnp.bfloat16)
    w2 = W2.T.astype(jnp.bfloat16)
    w3 = W3.T.astype(jnp.bfloat16)
    outs = []
    for s in range(NSLICE):
        lo = s * ES
        g0, g1 = _gather_slice(h, idx[0, lo:lo + ES], idx[1, lo:lo + ES])
        outs.append(_tc_mlp(
            g0, g1, w1a, w1b,
            b1.reshape(1, H1), w2, b2.reshape(1, H2), w3, b3.reshape(1, 1),
        ))
    return jnp.concatenate(outs, axis=0)
